# Initial kernel scaffold; baseline (speedup 1.0000x reference)
import jax
import jax.numpy as jnp
from jax.experimental import pallas as pl


def kernel(x, pos_w, pos_b, ln1_g, ln1_b, wq, wkv, wo, lepe_w, lepe_b, ln2_g, ln2_b, mlp_w1, mlp_b1, mlp_w2, mlp_b2):
    raise NotImplementedError("write your pallas kernel here")


# speedup vs baseline: 1.4771x; 1.4771x over previous
import functools

import jax
import jax.numpy as jnp
from jax.experimental import pallas as pl
from jax.experimental.pallas import tpu as pltpu

_B, _C, _H, _W = 2, 768, 56, 56
_NWIN = 7
_TOPK = 8
_NHEADS = 12
_HD = _C // _NHEADS
_P2 = _NWIN * _NWIN
_HW = (_H // _NWIN) * (_W // _NWIN)
_C4 = _C * 4
_NTOK = _B * _H * _W
_ROWS = 448
_NROW = _NTOK // _ROWS
_SR = 8


def _strip_conv(up, cur, dn, w_ref, pad, j, nstrip):
    ksize = 2 * pad + 1
    top = jnp.where(j > 0, up[_SR - pad:], jnp.zeros((pad, _W, _C), cur.dtype))
    bot = jnp.where(j < nstrip - 1, dn[:pad], jnp.zeros((pad, _W, _C), cur.dtype))
    xv = jnp.concatenate([top, cur, bot], axis=0)
    xp = jnp.pad(xv, ((0, 0), (pad, pad), (0, 0)))
    acc = jnp.zeros_like(cur)
    for dh in range(ksize):
        for dw in range(ksize):
            wv = w_ref[dh, dw, :].reshape(1, 1, _C)
            acc = acc + xp[dh:dh + _SR, dw:dw + _W, :] * wv
    return acc


def _dwconv_ln_kernel(xu_ref, xc_ref, xd_ref, w_ref, pb_ref, g_ref, b_ref,
                      y_ref, xn_ref):
    j = pl.program_id(1)
    acc = _strip_conv(xu_ref[0], xc_ref[0], xd_ref[0], w_ref, 1, j, _NWIN)
    y = xc_ref[0] + acc + pb_ref[0].reshape(1, 1, _C)
    y_ref[0] = y
    mu = jnp.mean(y, axis=-1, keepdims=True)
    var = jnp.mean((y - mu) ** 2, axis=-1, keepdims=True)
    xn = (y - mu) / jnp.sqrt(var + 1e-6)
    xn_ref[0] = xn * g_ref[0].reshape(1, 1, _C) + b_ref[0].reshape(1, 1, _C)


def _qkv_kernel(xw_ref, w_ref, q_ref, k_ref, v_ref, qm_ref, km_ref):
    xb = xw_ref[0].reshape(7 * _HW, _C)
    qkv = jnp.dot(xb, w_ref[...], preferred_element_type=jnp.float32)
    q = qkv[:, :_C]
    k = qkv[:, _C:2 * _C]
    v = qkv[:, 2 * _C:]
    q_ref[0] = q.reshape(7, _HW, _C)
    k_ref[0] = k.reshape(7, _HW, _C)
    v_ref[0] = v.reshape(7, _HW, _C)
    qm_ref[0] = q.reshape(7, _HW, _C).mean(axis=1, keepdims=True)
    km_ref[0] = k.reshape(7, _HW, _C).mean(axis=1, keepdims=True)


def _route_kernel(qm_ref, km_ref, idx_ref):
    qw = qm_ref[0, :, 0, :]
    kw = km_ref[0, :, 0, :]
    adj = jax.lax.dot_general(qw, kw, (((1,), (1,)), ((), ())),
                              preferred_element_type=jnp.float32)
    col = jax.lax.broadcasted_iota(jnp.int32, (_P2, _P2), 1)
    idxs = []
    a = adj
    for _ in range(_TOPK):
        m = jnp.max(a, axis=1, keepdims=True)
        idx = jnp.min(jnp.where(a >= m, col, _P2 * 2), axis=1)
        idxs.append(idx)
        a = jnp.where(col == idx[:, None], -jnp.inf, a)
    idx_ref[0] = jnp.stack(idxs, axis=1).astype(jnp.int32)


def _attn_kernel(idx_ref, q_ref, k_ref, v_ref, o_ref, kall_ref, vall_ref):
    t = pl.program_id(2)
    kall_ref[pl.ds(t * _HW, _HW), :] = k_ref[0, 0]
    vall_ref[pl.ds(t * _HW, _HW), :] = v_ref[0, 0]

    @pl.when(t == _TOPK - 1)
    def _():
        q = q_ref[0, 0]
        scale = _HD ** -0.5
        outs = []
        for h in range(_NHEADS):
            sl = slice(h * _HD, (h + 1) * _HD)
            qh = q[:, sl] * scale
            kh = kall_ref[:, sl]
            s = jax.lax.dot_general(qh, kh, (((1,), (1,)), ((), ())),
                                    preferred_element_type=jnp.float32)
            s = s - jnp.max(s, axis=1, keepdims=True)
            p = jnp.exp(s)
            p = p / jnp.sum(p, axis=1, keepdims=True)
            outs.append(jnp.dot(p, vall_ref[:, sl],
                                preferred_element_type=jnp.float32))
        o_ref[0, 0] = jnp.concatenate(outs, axis=1)


def _lepe_kernel(vu_ref, vc_ref, vd_ref, a_ref, w_ref, lb_ref, z_ref):
    j = pl.program_id(1)
    acc = _strip_conv(vu_ref[0], vc_ref[0], vd_ref[0], w_ref, 2, j, _NWIN)
    z_ref[0] = a_ref[0] + acc + lb_ref[0].reshape(1, 1, _C)


def _proj_kernel(z_ref, y_ref, w_ref, x2_ref):
    x2_ref[...] = y_ref[...] + jnp.dot(z_ref[...], w_ref[...],
                                       preferred_element_type=jnp.float32)


def _mlp_kernel(x_ref, g_ref, b_ref, w1_ref, b1_ref, w2_ref, b2_ref, o_ref):
    x = x_ref[...]
    mu = jnp.mean(x, axis=-1, keepdims=True)
    var = jnp.mean((x - mu) ** 2, axis=-1, keepdims=True)
    xn = (x - mu) / jnp.sqrt(var + 1e-6) * g_ref[0].reshape(1, _C) + b_ref[0].reshape(1, _C)
    h1 = jnp.dot(xn, w1_ref[...], preferred_element_type=jnp.float32) + b1_ref[0].reshape(1, _C4)
    h1 = 0.5 * h1 * (1.0 + jax.lax.erf(h1 * (2.0 ** -0.5)))
    o_ref[...] = x + jnp.dot(h1, w2_ref[...],
                             preferred_element_type=jnp.float32) + b2_ref[0].reshape(1, _C)


def kernel(x, pos_w, pos_b, ln1_g, ln1_b, wq, wkv, wo, lepe_w, lepe_b,
           ln2_g, ln2_b, mlp_w1, mlp_b1, mlp_w2, mlp_b2):
    f32 = jnp.float32
    x_bhwc = jnp.transpose(x, (0, 2, 3, 1))
    w3 = jnp.transpose(pos_w[:, 0], (1, 2, 0))
    w5 = jnp.transpose(lepe_w[:, 0], (1, 2, 0))

    _up = lambda b, j: (b, jnp.maximum(j - 1, 0), 0, 0)
    _cn = lambda b, j: (b, j, 0, 0)
    _dn = lambda b, j: (b, jnp.minimum(j + 1, _NWIN - 1), 0, 0)
    _strip = lambda: pl.BlockSpec((1, _SR, _W, _C), _cn)
    y, xn = pl.pallas_call(
        _dwconv_ln_kernel,
        grid=(_B, _NWIN),
        in_specs=[
            pl.BlockSpec((1, _SR, _W, _C), _up),
            pl.BlockSpec((1, _SR, _W, _C), _cn),
            pl.BlockSpec((1, _SR, _W, _C), _dn),
            pl.BlockSpec((3, 3, _C), lambda b, j: (0, 0, 0)),
            pl.BlockSpec((1, _C), lambda b, j: (0, 0)),
            pl.BlockSpec((1, _C), lambda b, j: (0, 0)),
            pl.BlockSpec((1, _C), lambda b, j: (0, 0)),
        ],
        out_specs=[_strip(), _strip()],
        out_shape=[
            jax.ShapeDtypeStruct((_B, _H, _W, _C), f32),
            jax.ShapeDtypeStruct((_B, _H, _W, _C), f32),
        ],
    )(x_bhwc, x_bhwc, x_bhwc, w3, pos_b.reshape(1, _C),
      ln1_g.reshape(1, _C), ln1_b.reshape(1, _C))

    xw = xn.reshape(_B, _NWIN, 8, _NWIN, 8, _C).transpose(0, 1, 3, 2, 4, 5)
    xw = xw.reshape(_B, _P2, _HW, _C)

    wqkv = jnp.concatenate([wq, wkv], axis=1)
    q, k, v, qm, km = pl.pallas_call(
        _qkv_kernel,
        grid=(_B, _P2 // 7),
        in_specs=[
            pl.BlockSpec((1, 7, _HW, _C), lambda b, j: (b, j, 0, 0)),
            pl.BlockSpec((_C, 3 * _C), lambda b, j: (0, 0)),
        ],
        out_specs=[
            pl.BlockSpec((1, 7, _HW, _C), lambda b, j: (b, j, 0, 0)),
            pl.BlockSpec((1, 7, _HW, _C), lambda b, j: (b, j, 0, 0)),
            pl.BlockSpec((1, 7, _HW, _C), lambda b, j: (b, j, 0, 0)),
            pl.BlockSpec((1, 7, 1, _C), lambda b, j: (b, j, 0, 0)),
            pl.BlockSpec((1, 7, 1, _C), lambda b, j: (b, j, 0, 0)),
        ],
        out_shape=[
            jax.ShapeDtypeStruct((_B, _P2, _HW, _C), f32),
            jax.ShapeDtypeStruct((_B, _P2, _HW, _C), f32),
            jax.ShapeDtypeStruct((_B, _P2, _HW, _C), f32),
            jax.ShapeDtypeStruct((_B, _P2, 1, _C), f32),
            jax.ShapeDtypeStruct((_B, _P2, 1, _C), f32),
        ],
    )(xw, wqkv)

    top_idx = pl.pallas_call(
        _route_kernel,
        grid=(_B,),
        in_specs=[
            pl.BlockSpec((1, _P2, 1, _C), lambda b: (b, 0, 0, 0)),
            pl.BlockSpec((1, _P2, 1, _C), lambda b: (b, 0, 0, 0)),
        ],
        out_specs=pl.BlockSpec((1, _P2, _TOPK), lambda b: (b, 0, 0)),
        out_shape=jax.ShapeDtypeStruct((_B, _P2, _TOPK), jnp.int32),
    )(qm, km)

    attn_out = pl.pallas_call(
        _attn_kernel,
        grid_spec=pltpu.PrefetchScalarGridSpec(
            num_scalar_prefetch=1,
            grid=(_B, _P2, _TOPK),
            in_specs=[
                pl.BlockSpec((1, 1, _HW, _C), lambda b, i, t, idx: (b, i, 0, 0)),
                pl.BlockSpec((1, 1, _HW, _C),
                             lambda b, i, t, idx: (b, idx[b, i, t], 0, 0)),
                pl.BlockSpec((1, 1, _HW, _C),
                             lambda b, i, t, idx: (b, idx[b, i, t], 0, 0)),
            ],
            out_specs=pl.BlockSpec((1, 1, _HW, _C), lambda b, i, t, idx: (b, i, 0, 0)),
            scratch_shapes=[
                pltpu.VMEM((_TOPK * _HW, _C), f32),
                pltpu.VMEM((_TOPK * _HW, _C), f32),
            ],
        ),
        out_shape=jax.ShapeDtypeStruct((_B, _P2, _HW, _C), f32),
    )(top_idx, q, k, v)

    def _reverse(t):
        t = t.reshape(_B, _NWIN, _NWIN, 8, 8, _C).transpose(0, 1, 3, 2, 4, 5)
        return t.reshape(_B, _H, _W, _C)

    attn_img = _reverse(attn_out)
    v_img = _reverse(v)

    z = pl.pallas_call(
        _lepe_kernel,
        grid=(_B, _NWIN),
        in_specs=[
            pl.BlockSpec((1, _SR, _W, _C), _up),
            pl.BlockSpec((1, _SR, _W, _C), _cn),
            pl.BlockSpec((1, _SR, _W, _C), _dn),
            pl.BlockSpec((1, _SR, _W, _C), _cn),
            pl.BlockSpec((5, 5, _C), lambda b, j: (0, 0, 0)),
            pl.BlockSpec((1, _C), lambda b, j: (0, 0)),
        ],
        out_specs=pl.BlockSpec((1, _SR, _W, _C), _cn),
        out_shape=jax.ShapeDtypeStruct((_B, _H, _W, _C), f32),
    )(v_img, v_img, v_img, attn_img, w5, lepe_b.reshape(1, _C))

    z_t = z.reshape(_NTOK, _C)
    y_t = y.reshape(_NTOK, _C)
    x2 = pl.pallas_call(
        _proj_kernel,
        grid=(_NROW,),
        in_specs=[
            pl.BlockSpec((_ROWS, _C), lambda r: (r, 0)),
            pl.BlockSpec((_ROWS, _C), lambda r: (r, 0)),
            pl.BlockSpec((_C, _C), lambda r: (0, 0)),
        ],
        out_specs=pl.BlockSpec((_ROWS, _C), lambda r: (r, 0)),
        out_shape=jax.ShapeDtypeStruct((_NTOK, _C), f32),
    )(z_t, y_t, wo)

    out_t = pl.pallas_call(
        _mlp_kernel,
        grid=(_NROW,),
        in_specs=[
            pl.BlockSpec((_ROWS, _C), lambda r: (r, 0)),
            pl.BlockSpec((1, _C), lambda r: (0, 0)),
            pl.BlockSpec((1, _C), lambda r: (0, 0)),
            pl.BlockSpec((_C, _C4), lambda r: (0, 0)),
            pl.BlockSpec((1, _C4), lambda r: (0, 0)),
            pl.BlockSpec((_C4, _C), lambda r: (0, 0)),
            pl.BlockSpec((1, _C), lambda r: (0, 0)),
        ],
        out_specs=pl.BlockSpec((_ROWS, _C), lambda r: (r, 0)),
        out_shape=jax.ShapeDtypeStruct((_NTOK, _C), f32),
    )(x2, ln2_g.reshape(1, _C), ln2_b.reshape(1, _C), mlp_w1,
      mlp_b1.reshape(1, _C4), mlp_w2, mlp_b2.reshape(1, _C))

    out = out_t.reshape(_B, _H, _W, _C)
    return jnp.transpose(out, (0, 3, 1, 2))


# speedup vs baseline: 1.5635x; 1.0584x over previous
import functools

import jax
import jax.numpy as jnp
from jax.experimental import pallas as pl
from jax.experimental.pallas import tpu as pltpu

_B, _C, _H, _W = 2, 768, 56, 56
_NWIN = 7
_TOPK = 8
_NHEADS = 12
_HD = _C // _NHEADS
_P2 = _NWIN * _NWIN
_HW = (_H // _NWIN) * (_W // _NWIN)
_C4 = _C * 4
_NTOK = _B * _H * _W
_ROWS = 448
_NROW = _NTOK // _ROWS
_SR = 8


def _strip_conv(up, cur, dn, w_ref, pad, j, nstrip):
    ksize = 2 * pad + 1
    top = jnp.where(j > 0, up[_SR - pad:], jnp.zeros((pad, _W, _C), cur.dtype))
    bot = jnp.where(j < nstrip - 1, dn[:pad], jnp.zeros((pad, _W, _C), cur.dtype))
    xv = jnp.concatenate([top, cur, bot], axis=0).astype(jnp.float32)
    xp = jnp.pad(xv, ((0, 0), (pad, pad), (0, 0)))
    acc = jnp.zeros((_SR, _W, _C), jnp.float32)
    for dh in range(ksize):
        for dw in range(ksize):
            wv = w_ref[dh, dw, :].reshape(1, 1, _C)
            acc = acc + xp[dh:dh + _SR, dw:dw + _W, :] * wv
    return acc


def _dwconv_ln_kernel(xu_ref, xc_ref, xd_ref, w_ref, pb_ref, g_ref, b_ref,
                      y_ref, xn_ref):
    j = pl.program_id(1)
    acc = _strip_conv(xu_ref[0], xc_ref[0], xd_ref[0], w_ref, 1, j, _NWIN)
    y = xc_ref[0] + acc + pb_ref[0].reshape(1, 1, _C)
    y_ref[0] = y
    mu = jnp.mean(y, axis=-1, keepdims=True)
    var = jnp.mean((y - mu) ** 2, axis=-1, keepdims=True)
    xn = (y - mu) / jnp.sqrt(var + 1e-6)
    xn = xn * g_ref[0].reshape(1, 1, _C) + b_ref[0].reshape(1, 1, _C)
    xn_ref[0] = xn.astype(jnp.bfloat16)


def _qkv_kernel(xw_ref, w_ref, q_ref, k_ref, v_ref, qm_ref, km_ref):
    xb = xw_ref[0].reshape(7 * _HW, _C)
    qkv = jnp.dot(xb, w_ref[...], preferred_element_type=jnp.float32)
    q = qkv[:, :_C]
    k = qkv[:, _C:2 * _C]
    v = qkv[:, 2 * _C:]
    q_ref[0] = q.reshape(7, _HW, _C).astype(jnp.bfloat16)
    k_ref[0] = k.reshape(7, _HW, _C).astype(jnp.bfloat16)
    v_ref[0] = v.reshape(7, _HW, _C).astype(jnp.bfloat16)
    qm_ref[0] = q.reshape(7, _HW, _C).mean(axis=1, keepdims=True)
    km_ref[0] = k.reshape(7, _HW, _C).mean(axis=1, keepdims=True)


def _route_kernel(qm_ref, km_ref, idx_ref):
    qw = qm_ref[0, :, 0, :]
    kw = km_ref[0, :, 0, :]
    adj = jax.lax.dot_general(qw, kw, (((1,), (1,)), ((), ())),
                              preferred_element_type=jnp.float32)
    col = jax.lax.broadcasted_iota(jnp.int32, (_P2, _P2), 1)
    idxs = []
    a = adj
    for _ in range(_TOPK):
        m = jnp.max(a, axis=1, keepdims=True)
        idx = jnp.min(jnp.where(a >= m, col, _P2 * 2), axis=1)
        idxs.append(idx)
        a = jnp.where(col == idx[:, None], -jnp.inf, a)
    idx_ref[0] = jnp.stack(idxs, axis=1).astype(jnp.int32)


def _attn_kernel(idx_ref, q_ref, k_ref, v_ref, o_ref, kall_ref, vall_ref):
    t = pl.program_id(2)
    kall_ref[pl.ds(t * _HW, _HW), :] = k_ref[0, 0]
    vall_ref[pl.ds(t * _HW, _HW), :] = v_ref[0, 0]

    @pl.when(t == _TOPK - 1)
    def _():
        q = q_ref[0, 0]
        scale = _HD ** -0.5
        outs = []
        for h in range(_NHEADS):
            sl = slice(h * _HD, (h + 1) * _HD)
            qh = q[:, sl]
            kh = kall_ref[:, sl]
            s = jax.lax.dot_general(qh, kh, (((1,), (1,)), ((), ())),
                                    preferred_element_type=jnp.float32) * scale
            s = s - jnp.max(s, axis=1, keepdims=True)
            p = jnp.exp(s)
            p = (p / jnp.sum(p, axis=1, keepdims=True)).astype(jnp.bfloat16)
            outs.append(jnp.dot(p, vall_ref[:, sl],
                                preferred_element_type=jnp.float32))
        o_ref[0, 0] = jnp.concatenate(outs, axis=1).astype(jnp.bfloat16)


def _lepe_kernel(vu_ref, vc_ref, vd_ref, a_ref, w_ref, lb_ref, z_ref):
    j = pl.program_id(1)
    acc = _strip_conv(vu_ref[0], vc_ref[0], vd_ref[0], w_ref, 2, j, _NWIN)
    z = a_ref[0].astype(jnp.float32) + acc + lb_ref[0].reshape(1, 1, _C)
    z_ref[0] = z.astype(jnp.bfloat16)


def _proj_kernel(z_ref, y_ref, w_ref, x2_ref):
    x2_ref[...] = y_ref[...] + jnp.dot(z_ref[...], w_ref[...],
                                       preferred_element_type=jnp.float32)


def _mlp_kernel(x_ref, g_ref, b_ref, w1_ref, b1_ref, w2_ref, b2_ref, o_ref):
    x = x_ref[...]
    mu = jnp.mean(x, axis=-1, keepdims=True)
    var = jnp.mean((x - mu) ** 2, axis=-1, keepdims=True)
    xn = (x - mu) / jnp.sqrt(var + 1e-6) * g_ref[0].reshape(1, _C) + b_ref[0].reshape(1, _C)
    h1 = jnp.dot(xn.astype(jnp.bfloat16), w1_ref[...],
                 preferred_element_type=jnp.float32) + b1_ref[0].reshape(1, _C4)
    h1 = 0.5 * h1 * (1.0 + jax.lax.erf(h1 * (2.0 ** -0.5)))
    o_ref[...] = x + jnp.dot(h1.astype(jnp.bfloat16), w2_ref[...],
                             preferred_element_type=jnp.float32) + b2_ref[0].reshape(1, _C)


def kernel(x, pos_w, pos_b, ln1_g, ln1_b, wq, wkv, wo, lepe_w, lepe_b,
           ln2_g, ln2_b, mlp_w1, mlp_b1, mlp_w2, mlp_b2):
    f32 = jnp.float32
    x_bhwc = jnp.transpose(x, (0, 2, 3, 1))
    w3 = jnp.transpose(pos_w[:, 0], (1, 2, 0))
    w5 = jnp.transpose(lepe_w[:, 0], (1, 2, 0))

    _up = lambda b, j: (b, jnp.maximum(j - 1, 0), 0, 0)
    _cn = lambda b, j: (b, j, 0, 0)
    _dn = lambda b, j: (b, jnp.minimum(j + 1, _NWIN - 1), 0, 0)
    _strip = lambda: pl.BlockSpec((1, _SR, _W, _C), _cn)
    y, xn = pl.pallas_call(
        _dwconv_ln_kernel,
        grid=(_B, _NWIN),
        in_specs=[
            pl.BlockSpec((1, _SR, _W, _C), _up),
            pl.BlockSpec((1, _SR, _W, _C), _cn),
            pl.BlockSpec((1, _SR, _W, _C), _dn),
            pl.BlockSpec((3, 3, _C), lambda b, j: (0, 0, 0)),
            pl.BlockSpec((1, _C), lambda b, j: (0, 0)),
            pl.BlockSpec((1, _C), lambda b, j: (0, 0)),
            pl.BlockSpec((1, _C), lambda b, j: (0, 0)),
        ],
        out_specs=[_strip(), _strip()],
        out_shape=[
            jax.ShapeDtypeStruct((_B, _H, _W, _C), f32),
            jax.ShapeDtypeStruct((_B, _H, _W, _C), jnp.bfloat16),
        ],
    )(x_bhwc, x_bhwc, x_bhwc, w3, pos_b.reshape(1, _C),
      ln1_g.reshape(1, _C), ln1_b.reshape(1, _C))

    xw = xn.reshape(_B, _NWIN, 8, _NWIN, 8, _C).transpose(0, 1, 3, 2, 4, 5)
    xw = xw.reshape(_B, _P2, _HW, _C)

    wqkv = jnp.concatenate([wq, wkv], axis=1).astype(jnp.bfloat16)
    q, k, v, qm, km = pl.pallas_call(
        _qkv_kernel,
        grid=(_B, _P2 // 7),
        in_specs=[
            pl.BlockSpec((1, 7, _HW, _C), lambda b, j: (b, j, 0, 0)),
            pl.BlockSpec((_C, 3 * _C), lambda b, j: (0, 0)),
        ],
        out_specs=[
            pl.BlockSpec((1, 7, _HW, _C), lambda b, j: (b, j, 0, 0)),
            pl.BlockSpec((1, 7, _HW, _C), lambda b, j: (b, j, 0, 0)),
            pl.BlockSpec((1, 7, _HW, _C), lambda b, j: (b, j, 0, 0)),
            pl.BlockSpec((1, 7, 1, _C), lambda b, j: (b, j, 0, 0)),
            pl.BlockSpec((1, 7, 1, _C), lambda b, j: (b, j, 0, 0)),
        ],
        out_shape=[
            jax.ShapeDtypeStruct((_B, _P2, _HW, _C), jnp.bfloat16),
            jax.ShapeDtypeStruct((_B, _P2, _HW, _C), jnp.bfloat16),
            jax.ShapeDtypeStruct((_B, _P2, _HW, _C), jnp.bfloat16),
            jax.ShapeDtypeStruct((_B, _P2, 1, _C), f32),
            jax.ShapeDtypeStruct((_B, _P2, 1, _C), f32),
        ],
    )(xw, wqkv)

    top_idx = pl.pallas_call(
        _route_kernel,
        grid=(_B,),
        in_specs=[
            pl.BlockSpec((1, _P2, 1, _C), lambda b: (b, 0, 0, 0)),
            pl.BlockSpec((1, _P2, 1, _C), lambda b: (b, 0, 0, 0)),
        ],
        out_specs=pl.BlockSpec((1, _P2, _TOPK), lambda b: (b, 0, 0)),
        out_shape=jax.ShapeDtypeStruct((_B, _P2, _TOPK), jnp.int32),
    )(qm, km)

    attn_out = pl.pallas_call(
        _attn_kernel,
        grid_spec=pltpu.PrefetchScalarGridSpec(
            num_scalar_prefetch=1,
            grid=(_B, _P2, _TOPK),
            in_specs=[
                pl.BlockSpec((1, 1, _HW, _C), lambda b, i, t, idx: (b, i, 0, 0)),
                pl.BlockSpec((1, 1, _HW, _C),
                             lambda b, i, t, idx: (b, idx[b, i, t], 0, 0)),
                pl.BlockSpec((1, 1, _HW, _C),
                             lambda b, i, t, idx: (b, idx[b, i, t], 0, 0)),
            ],
            out_specs=pl.BlockSpec((1, 1, _HW, _C), lambda b, i, t, idx: (b, i, 0, 0)),
            scratch_shapes=[
                pltpu.VMEM((_TOPK * _HW, _C), jnp.bfloat16),
                pltpu.VMEM((_TOPK * _HW, _C), jnp.bfloat16),
            ],
        ),
        out_shape=jax.ShapeDtypeStruct((_B, _P2, _HW, _C), jnp.bfloat16),
    )(top_idx, q, k, v)

    def _reverse(t):
        t = t.reshape(_B, _NWIN, _NWIN, 8, 8, _C).transpose(0, 1, 3, 2, 4, 5)
        return t.reshape(_B, _H, _W, _C)

    attn_img = _reverse(attn_out)
    v_img = _reverse(v)

    z = pl.pallas_call(
        _lepe_kernel,
        grid=(_B, _NWIN),
        in_specs=[
            pl.BlockSpec((1, _SR, _W, _C), _up),
            pl.BlockSpec((1, _SR, _W, _C), _cn),
            pl.BlockSpec((1, _SR, _W, _C), _dn),
            pl.BlockSpec((1, _SR, _W, _C), _cn),
            pl.BlockSpec((5, 5, _C), lambda b, j: (0, 0, 0)),
            pl.BlockSpec((1, _C), lambda b, j: (0, 0)),
        ],
        out_specs=pl.BlockSpec((1, _SR, _W, _C), _cn),
        out_shape=jax.ShapeDtypeStruct((_B, _H, _W, _C), jnp.bfloat16),
    )(v_img, v_img, v_img, attn_img, w5, lepe_b.reshape(1, _C))

    z_t = z.reshape(_NTOK, _C)
    y_t = y.reshape(_NTOK, _C)
    x2 = pl.pallas_call(
        _proj_kernel,
        grid=(_NROW,),
        in_specs=[
            pl.BlockSpec((_ROWS, _C), lambda r: (r, 0)),
            pl.BlockSpec((_ROWS, _C), lambda r: (r, 0)),
            pl.BlockSpec((_C, _C), lambda r: (0, 0)),
        ],
        out_specs=pl.BlockSpec((_ROWS, _C), lambda r: (r, 0)),
        out_shape=jax.ShapeDtypeStruct((_NTOK, _C), f32),
    )(z_t, y_t, wo.astype(jnp.bfloat16))

    out_t = pl.pallas_call(
        _mlp_kernel,
        grid=(_NROW,),
        in_specs=[
            pl.BlockSpec((_ROWS, _C), lambda r: (r, 0)),
            pl.BlockSpec((1, _C), lambda r: (0, 0)),
            pl.BlockSpec((1, _C), lambda r: (0, 0)),
            pl.BlockSpec((_C, _C4), lambda r: (0, 0)),
            pl.BlockSpec((1, _C4), lambda r: (0, 0)),
            pl.BlockSpec((_C4, _C), lambda r: (0, 0)),
            pl.BlockSpec((1, _C), lambda r: (0, 0)),
        ],
        out_specs=pl.BlockSpec((_ROWS, _C), lambda r: (r, 0)),
        out_shape=jax.ShapeDtypeStruct((_NTOK, _C), f32),
    )(x2, ln2_g.reshape(1, _C), ln2_b.reshape(1, _C), mlp_w1.astype(jnp.bfloat16),
      mlp_b1.reshape(1, _C4), mlp_w2.astype(jnp.bfloat16), mlp_b2.reshape(1, _C))

    out = out_t.reshape(_B, _H, _W, _C)
    return jnp.transpose(out, (0, 3, 1, 2))


# speedup vs baseline: 2.3414x; 1.4976x over previous
import functools

import jax
import jax.numpy as jnp
from jax.experimental import pallas as pl
from jax.experimental.pallas import tpu as pltpu

_B, _C, _H, _W = 2, 768, 56, 56
_NWIN = 7
_TOPK = 8
_NHEADS = 12
_HD = _C // _NHEADS
_P2 = _NWIN * _NWIN
_HW = (_H // _NWIN) * (_W // _NWIN)
_C4 = _C * 4
_NTOK = _B * _H * _W
_ROWS = 448
_NROW = _NTOK // _ROWS
_SR = 8


def _strip_conv(up, cur, dn, w_ref, pad, j, nstrip):
    ksize = 2 * pad + 1
    top = jnp.where(j > 0, up[_SR - pad:], jnp.zeros((pad, _W, _C), cur.dtype))
    bot = jnp.where(j < nstrip - 1, dn[:pad], jnp.zeros((pad, _W, _C), cur.dtype))
    xv = jnp.concatenate([top, cur, bot], axis=0).astype(jnp.float32)
    xp = jnp.pad(xv, ((0, 0), (pad, pad), (0, 0)))
    acc = jnp.zeros((_SR, _W, _C), jnp.float32)
    for dh in range(ksize):
        for dw in range(ksize):
            wv = w_ref[dh, dw, :].reshape(1, 1, _C)
            acc = acc + xp[dh:dh + _SR, dw:dw + _W, :] * wv
    return acc


def _dwconv_ln_kernel(xu_ref, xc_ref, xd_ref, w_ref, pb_ref, g_ref, b_ref,
                      y_ref, xn_ref):
    j = pl.program_id(1)
    acc = _strip_conv(xu_ref[0], xc_ref[0], xd_ref[0], w_ref, 1, j, _NWIN)
    y = xc_ref[0] + acc + pb_ref[0].reshape(1, 1, _C)
    y_ref[0] = y
    mu = jnp.mean(y, axis=-1, keepdims=True)
    var = jnp.mean((y - mu) ** 2, axis=-1, keepdims=True)
    xn = (y - mu) / jnp.sqrt(var + 1e-6)
    xn = xn * g_ref[0].reshape(1, 1, _C) + b_ref[0].reshape(1, 1, _C)
    xn_ref[0] = xn.astype(jnp.bfloat16)


def _qkv_kernel(xw_ref, w_ref, q_ref, k_ref, v_ref, qm_ref, km_ref):
    xb = xw_ref[0].reshape(7 * _HW, _C)
    qkv = jnp.dot(xb, w_ref[...], preferred_element_type=jnp.float32)
    q = qkv[:, :_C]
    k = qkv[:, _C:2 * _C]
    v = qkv[:, 2 * _C:]
    q_ref[0] = q.reshape(7, _HW, _C).astype(jnp.bfloat16)
    k_ref[0] = k.reshape(7, _HW, _C).astype(jnp.bfloat16)
    v_ref[0] = v.reshape(7, _HW, _C).astype(jnp.bfloat16)
    qm_ref[0] = q.reshape(7, _HW, _C).mean(axis=1, keepdims=True)
    km_ref[0] = k.reshape(7, _HW, _C).mean(axis=1, keepdims=True)


def _route_kernel(qm_ref, km_ref, idx_ref):
    qw = qm_ref[0, :, 0, :]
    kw = km_ref[0, :, 0, :]
    adj = jax.lax.dot_general(qw, kw, (((1,), (1,)), ((), ())),
                              preferred_element_type=jnp.float32)
    col = jax.lax.broadcasted_iota(jnp.int32, (_P2, _P2), 1)
    idxs = []
    a = adj
    for _ in range(_TOPK):
        m = jnp.max(a, axis=1, keepdims=True)
        idx = jnp.min(jnp.where(a >= m, col, _P2 * 2), axis=1)
        idxs.append(idx)
        a = jnp.where(col == idx[:, None], -jnp.inf, a)
    idx_ref[0] = jnp.stack(idxs, axis=1).astype(jnp.int32)


def _attn_kernel(idx_ref, q_ref, k_ref, v_ref, o_ref, kall_ref, vall_ref):
    b = pl.program_id(0)
    i = pl.program_id(1)
    for t in range(_TOPK):
        w = idx_ref[b, i, t]
        kall_ref[pl.ds(t * _HW, _HW), :] = k_ref[0, w]
        vall_ref[pl.ds(t * _HW, _HW), :] = v_ref[0, w]
    q = q_ref[0, 0]
    scale = _HD ** -0.5
    outs = []
    for h in range(_NHEADS):
        sl = slice(h * _HD, (h + 1) * _HD)
        qh = q[:, sl]
        kh = kall_ref[:, sl]
        s = jax.lax.dot_general(qh, kh, (((1,), (1,)), ((), ())),
                                preferred_element_type=jnp.float32) * scale
        s = s - jnp.max(s, axis=1, keepdims=True)
        p = jnp.exp(s)
        p = (p / jnp.sum(p, axis=1, keepdims=True)).astype(jnp.bfloat16)
        outs.append(jnp.dot(p, vall_ref[:, sl],
                            preferred_element_type=jnp.float32))
    o_ref[0, 0] = jnp.concatenate(outs, axis=1).astype(jnp.bfloat16)


def _lepe_kernel(vu_ref, vc_ref, vd_ref, a_ref, w_ref, lb_ref, z_ref):
    j = pl.program_id(1)
    acc = _strip_conv(vu_ref[0], vc_ref[0], vd_ref[0], w_ref, 2, j, _NWIN)
    z = a_ref[0].astype(jnp.float32) + acc + lb_ref[0].reshape(1, 1, _C)
    z_ref[0] = z.astype(jnp.bfloat16)


def _proj_kernel(z_ref, y_ref, w_ref, x2_ref):
    x2_ref[...] = y_ref[...] + jnp.dot(z_ref[...], w_ref[...],
                                       preferred_element_type=jnp.float32)


def _mlp_kernel(x_ref, g_ref, b_ref, w1_ref, b1_ref, w2_ref, b2_ref, o_ref):
    x = x_ref[...]
    mu = jnp.mean(x, axis=-1, keepdims=True)
    var = jnp.mean((x - mu) ** 2, axis=-1, keepdims=True)
    xn = (x - mu) / jnp.sqrt(var + 1e-6) * g_ref[0].reshape(1, _C) + b_ref[0].reshape(1, _C)
    h1 = jnp.dot(xn.astype(jnp.bfloat16), w1_ref[...],
                 preferred_element_type=jnp.float32) + b1_ref[0].reshape(1, _C4)
    h1 = 0.5 * h1 * (1.0 + jax.lax.erf(h1 * (2.0 ** -0.5)))
    o_ref[...] = x + jnp.dot(h1.astype(jnp.bfloat16), w2_ref[...],
                             preferred_element_type=jnp.float32) + b2_ref[0].reshape(1, _C)


def kernel(x, pos_w, pos_b, ln1_g, ln1_b, wq, wkv, wo, lepe_w, lepe_b,
           ln2_g, ln2_b, mlp_w1, mlp_b1, mlp_w2, mlp_b2):
    f32 = jnp.float32
    x_bhwc = jnp.transpose(x, (0, 2, 3, 1))
    w3 = jnp.transpose(pos_w[:, 0], (1, 2, 0))
    w5 = jnp.transpose(lepe_w[:, 0], (1, 2, 0))

    _up = lambda b, j: (b, jnp.maximum(j - 1, 0), 0, 0)
    _cn = lambda b, j: (b, j, 0, 0)
    _dn = lambda b, j: (b, jnp.minimum(j + 1, _NWIN - 1), 0, 0)
    _strip = lambda: pl.BlockSpec((1, _SR, _W, _C), _cn)
    y, xn = pl.pallas_call(
        _dwconv_ln_kernel,
        grid=(_B, _NWIN),
        in_specs=[
            pl.BlockSpec((1, _SR, _W, _C), _up),
            pl.BlockSpec((1, _SR, _W, _C), _cn),
            pl.BlockSpec((1, _SR, _W, _C), _dn),
            pl.BlockSpec((3, 3, _C), lambda b, j: (0, 0, 0)),
            pl.BlockSpec((1, _C), lambda b, j: (0, 0)),
            pl.BlockSpec((1, _C), lambda b, j: (0, 0)),
            pl.BlockSpec((1, _C), lambda b, j: (0, 0)),
        ],
        out_specs=[_strip(), _strip()],
        out_shape=[
            jax.ShapeDtypeStruct((_B, _H, _W, _C), f32),
            jax.ShapeDtypeStruct((_B, _H, _W, _C), jnp.bfloat16),
        ],
    )(x_bhwc, x_bhwc, x_bhwc, w3, pos_b.reshape(1, _C),
      ln1_g.reshape(1, _C), ln1_b.reshape(1, _C))

    xw = xn.reshape(_B, _NWIN, 8, _NWIN, 8, _C).transpose(0, 1, 3, 2, 4, 5)
    xw = xw.reshape(_B, _P2, _HW, _C)

    wqkv = jnp.concatenate([wq, wkv], axis=1).astype(jnp.bfloat16)
    q, k, v, qm, km = pl.pallas_call(
        _qkv_kernel,
        grid=(_B, _P2 // 7),
        in_specs=[
            pl.BlockSpec((1, 7, _HW, _C), lambda b, j: (b, j, 0, 0)),
            pl.BlockSpec((_C, 3 * _C), lambda b, j: (0, 0)),
        ],
        out_specs=[
            pl.BlockSpec((1, 7, _HW, _C), lambda b, j: (b, j, 0, 0)),
            pl.BlockSpec((1, 7, _HW, _C), lambda b, j: (b, j, 0, 0)),
            pl.BlockSpec((1, 7, _HW, _C), lambda b, j: (b, j, 0, 0)),
            pl.BlockSpec((1, 7, 1, _C), lambda b, j: (b, j, 0, 0)),
            pl.BlockSpec((1, 7, 1, _C), lambda b, j: (b, j, 0, 0)),
        ],
        out_shape=[
            jax.ShapeDtypeStruct((_B, _P2, _HW, _C), jnp.bfloat16),
            jax.ShapeDtypeStruct((_B, _P2, _HW, _C), jnp.bfloat16),
            jax.ShapeDtypeStruct((_B, _P2, _HW, _C), jnp.bfloat16),
            jax.ShapeDtypeStruct((_B, _P2, 1, _C), f32),
            jax.ShapeDtypeStruct((_B, _P2, 1, _C), f32),
        ],
    )(xw, wqkv)

    top_idx = pl.pallas_call(
        _route_kernel,
        grid=(_B,),
        in_specs=[
            pl.BlockSpec((1, _P2, 1, _C), lambda b: (b, 0, 0, 0)),
            pl.BlockSpec((1, _P2, 1, _C), lambda b: (b, 0, 0, 0)),
        ],
        out_specs=pl.BlockSpec((1, _P2, _TOPK), lambda b: (b, 0, 0)),
        out_shape=jax.ShapeDtypeStruct((_B, _P2, _TOPK), jnp.int32),
    )(qm, km)

    attn_out = pl.pallas_call(
        _attn_kernel,
        grid_spec=pltpu.PrefetchScalarGridSpec(
            num_scalar_prefetch=1,
            grid=(_B, _P2),
            in_specs=[
                pl.BlockSpec((1, 1, _HW, _C), lambda b, i, idx: (b, i, 0, 0)),
                pl.BlockSpec((1, _P2, _HW, _C), lambda b, i, idx: (b, 0, 0, 0)),
                pl.BlockSpec((1, _P2, _HW, _C), lambda b, i, idx: (b, 0, 0, 0)),
            ],
            out_specs=pl.BlockSpec((1, 1, _HW, _C), lambda b, i, idx: (b, i, 0, 0)),
            scratch_shapes=[
                pltpu.VMEM((_TOPK * _HW, _C), jnp.bfloat16),
                pltpu.VMEM((_TOPK * _HW, _C), jnp.bfloat16),
            ],
        ),
        out_shape=jax.ShapeDtypeStruct((_B, _P2, _HW, _C), jnp.bfloat16),
    )(top_idx, q, k, v)

    def _reverse(t):
        t = t.reshape(_B, _NWIN, _NWIN, 8, 8, _C).transpose(0, 1, 3, 2, 4, 5)
        return t.reshape(_B, _H, _W, _C)

    attn_img = _reverse(attn_out)
    v_img = _reverse(v)

    z = pl.pallas_call(
        _lepe_kernel,
        grid=(_B, _NWIN),
        in_specs=[
            pl.BlockSpec((1, _SR, _W, _C), _up),
            pl.BlockSpec((1, _SR, _W, _C), _cn),
            pl.BlockSpec((1, _SR, _W, _C), _dn),
            pl.BlockSpec((1, _SR, _W, _C), _cn),
            pl.BlockSpec((5, 5, _C), lambda b, j: (0, 0, 0)),
            pl.BlockSpec((1, _C), lambda b, j: (0, 0)),
        ],
        out_specs=pl.BlockSpec((1, _SR, _W, _C), _cn),
        out_shape=jax.ShapeDtypeStruct((_B, _H, _W, _C), jnp.bfloat16),
    )(v_img, v_img, v_img, attn_img, w5, lepe_b.reshape(1, _C))

    z_t = z.reshape(_NTOK, _C)
    y_t = y.reshape(_NTOK, _C)
    x2 = pl.pallas_call(
        _proj_kernel,
        grid=(_NROW,),
        in_specs=[
            pl.BlockSpec((_ROWS, _C), lambda r: (r, 0)),
            pl.BlockSpec((_ROWS, _C), lambda r: (r, 0)),
            pl.BlockSpec((_C, _C), lambda r: (0, 0)),
        ],
        out_specs=pl.BlockSpec((_ROWS, _C), lambda r: (r, 0)),
        out_shape=jax.ShapeDtypeStruct((_NTOK, _C), f32),
    )(z_t, y_t, wo.astype(jnp.bfloat16))

    out_t = pl.pallas_call(
        _mlp_kernel,
        grid=(_NROW,),
        in_specs=[
            pl.BlockSpec((_ROWS, _C), lambda r: (r, 0)),
            pl.BlockSpec((1, _C), lambda r: (0, 0)),
            pl.BlockSpec((1, _C), lambda r: (0, 0)),
            pl.BlockSpec((_C, _C4), lambda r: (0, 0)),
            pl.BlockSpec((1, _C4), lambda r: (0, 0)),
            pl.BlockSpec((_C4, _C), lambda r: (0, 0)),
            pl.BlockSpec((1, _C), lambda r: (0, 0)),
        ],
        out_specs=pl.BlockSpec((_ROWS, _C), lambda r: (r, 0)),
        out_shape=jax.ShapeDtypeStruct((_NTOK, _C), f32),
    )(x2, ln2_g.reshape(1, _C), ln2_b.reshape(1, _C), mlp_w1.astype(jnp.bfloat16),
      mlp_b1.reshape(1, _C4), mlp_w2.astype(jnp.bfloat16), mlp_b2.reshape(1, _C))

    out = out_t.reshape(_B, _H, _W, _C)
    return jnp.transpose(out, (0, 3, 1, 2))


# speedup vs baseline: 2.4216x; 1.0343x over previous
import functools

import jax
import jax.numpy as jnp
from jax.experimental import pallas as pl
from jax.experimental.pallas import tpu as pltpu

_B, _C, _H, _W = 2, 768, 56, 56
_NWIN = 7
_TOPK = 8
_NHEADS = 12
_HD = _C // _NHEADS
_P2 = _NWIN * _NWIN
_HW = (_H // _NWIN) * (_W // _NWIN)
_C4 = _C * 4
_NTOK = _B * _H * _W
_ROWS = 448
_NROW = _NTOK // _ROWS
_SR = 8


def _strip_conv(up, cur, dn, w_ref, pad, j, nstrip):
    ksize = 2 * pad + 1
    top = jnp.where(j > 0, up[_SR - pad:], jnp.zeros((pad, _W, _C), cur.dtype))
    bot = jnp.where(j < nstrip - 1, dn[:pad], jnp.zeros((pad, _W, _C), cur.dtype))
    xv = jnp.concatenate([top, cur, bot], axis=0).astype(jnp.float32)
    xp = jnp.pad(xv, ((0, 0), (pad, pad), (0, 0)))
    acc = jnp.zeros((_SR, _W, _C), jnp.float32)
    for dh in range(ksize):
        for dw in range(ksize):
            wv = w_ref[dh, dw, :].reshape(1, 1, _C)
            acc = acc + xp[dh:dh + _SR, dw:dw + _W, :] * wv
    return acc


def _dwconv_ln_kernel(xu_ref, xc_ref, xd_ref, w_ref, pb_ref, g_ref, b_ref,
                      y_ref, xn_ref):
    j = pl.program_id(1)
    acc = _strip_conv(xu_ref[0], xc_ref[0], xd_ref[0], w_ref, 1, j, _NWIN)
    y = xc_ref[0] + acc + pb_ref[0].reshape(1, 1, _C)
    y_ref[0] = y
    mu = jnp.mean(y, axis=-1, keepdims=True)
    var = jnp.mean((y - mu) ** 2, axis=-1, keepdims=True)
    xn = (y - mu) / jnp.sqrt(var + 1e-6)
    xn = xn * g_ref[0].reshape(1, 1, _C) + b_ref[0].reshape(1, 1, _C)
    xn_ref[0] = xn.astype(jnp.bfloat16)


def _qkv_kernel(xw_ref, w_ref, q_ref, k_ref, v_ref, qm_ref, km_ref):
    xb = xw_ref[0].reshape(7 * _HW, _C)
    qkv = jnp.dot(xb, w_ref[...], preferred_element_type=jnp.float32)
    q = qkv[:, :_C]
    k = qkv[:, _C:2 * _C]
    v = qkv[:, 2 * _C:]
    q_ref[0] = q.reshape(7, _HW, _C).astype(jnp.bfloat16)
    k_ref[0] = k.reshape(7, _HW, _C).astype(jnp.bfloat16)
    v_ref[0] = v.reshape(7, _HW, _C).astype(jnp.bfloat16)
    qm_ref[0] = q.reshape(7, _HW, _C).mean(axis=1, keepdims=True)
    km_ref[0] = k.reshape(7, _HW, _C).mean(axis=1, keepdims=True)


def _route_kernel(qm_ref, km_ref, idx_ref):
    qw = qm_ref[0, :, 0, :]
    kw = km_ref[0, :, 0, :]
    adj = jax.lax.dot_general(qw, kw, (((1,), (1,)), ((), ())),
                              preferred_element_type=jnp.float32)
    col = jax.lax.broadcasted_iota(jnp.int32, (_P2, _P2), 1)
    idxs = []
    a = adj
    for _ in range(_TOPK):
        m = jnp.max(a, axis=1, keepdims=True)
        idx = jnp.min(jnp.where(a >= m, col, _P2 * 2), axis=1)
        idxs.append(idx)
        a = jnp.where(col == idx[:, None], -jnp.inf, a)
    idx_ref[0] = jnp.stack(idxs, axis=1).astype(jnp.int32)


def _attn_kernel(idx_ref, q_ref, k_ref, v_ref, o_ref, kall_ref, vall_ref,
                 s_ref, p_ref, sems):
    b = pl.program_id(0)
    i = pl.program_id(1)
    copies = []
    for t in range(_TOPK):
        w = idx_ref[b, i, t]
        for src, dst, s in ((k_ref, kall_ref, 0), (v_ref, vall_ref, 1)):
            c = pltpu.make_async_copy(
                src.at[0, w], dst.at[pl.ds(t * _HW, _HW), :], sems.at[2 * t + s])
            c.start()
            copies.append(c)
    for c in copies:
        c.wait()
    q = q_ref[0, 0]
    scale = _HD ** -0.5
    for h in range(_NHEADS):
        sl = slice(h * _HD, (h + 1) * _HD)
        s_ref[pl.ds(h * _HW, _HW), :] = jax.lax.dot_general(
            q[:, sl], kall_ref[:, sl], (((1,), (1,)), ((), ())),
            preferred_element_type=jnp.float32)
    s = s_ref[...] * scale
    s = s - jnp.max(s, axis=1, keepdims=True)
    e = jnp.exp(s)
    p_ref[...] = (e / jnp.sum(e, axis=1, keepdims=True)).astype(jnp.bfloat16)
    outs = []
    for h in range(_NHEADS):
        sl = slice(h * _HD, (h + 1) * _HD)
        outs.append(jnp.dot(p_ref[pl.ds(h * _HW, _HW), :], vall_ref[:, sl],
                            preferred_element_type=jnp.float32))
    o_ref[0, 0] = jnp.concatenate(outs, axis=1).astype(jnp.bfloat16)


def _lepe_kernel(vu_ref, vc_ref, vd_ref, a_ref, w_ref, lb_ref, z_ref):
    j = pl.program_id(1)
    acc = _strip_conv(vu_ref[0], vc_ref[0], vd_ref[0], w_ref, 2, j, _NWIN)
    z = a_ref[0].astype(jnp.float32) + acc + lb_ref[0].reshape(1, 1, _C)
    z_ref[0] = z.astype(jnp.bfloat16)


def _proj_kernel(z_ref, y_ref, w_ref, x2_ref):
    x2_ref[...] = y_ref[...] + jnp.dot(z_ref[...], w_ref[...],
                                       preferred_element_type=jnp.float32)


def _mlp_kernel(x_ref, g_ref, b_ref, w1_ref, b1_ref, w2_ref, b2_ref, o_ref):
    x = x_ref[...]
    mu = jnp.mean(x, axis=-1, keepdims=True)
    var = jnp.mean((x - mu) ** 2, axis=-1, keepdims=True)
    xn = (x - mu) / jnp.sqrt(var + 1e-6) * g_ref[0].reshape(1, _C) + b_ref[0].reshape(1, _C)
    h1 = jnp.dot(xn.astype(jnp.bfloat16), w1_ref[...],
                 preferred_element_type=jnp.float32) + b1_ref[0].reshape(1, _C4)
    h1 = 0.5 * h1 * (1.0 + jax.lax.erf(h1 * (2.0 ** -0.5)))
    o_ref[...] = x + jnp.dot(h1.astype(jnp.bfloat16), w2_ref[...],
                             preferred_element_type=jnp.float32) + b2_ref[0].reshape(1, _C)


def kernel(x, pos_w, pos_b, ln1_g, ln1_b, wq, wkv, wo, lepe_w, lepe_b,
           ln2_g, ln2_b, mlp_w1, mlp_b1, mlp_w2, mlp_b2):
    f32 = jnp.float32
    x_bhwc = jnp.transpose(x, (0, 2, 3, 1))
    w3 = jnp.transpose(pos_w[:, 0], (1, 2, 0))
    w5 = jnp.transpose(lepe_w[:, 0], (1, 2, 0))

    _up = lambda b, j: (b, jnp.maximum(j - 1, 0), 0, 0)
    _cn = lambda b, j: (b, j, 0, 0)
    _dn = lambda b, j: (b, jnp.minimum(j + 1, _NWIN - 1), 0, 0)
    _strip = lambda: pl.BlockSpec((1, _SR, _W, _C), _cn)
    y, xn = pl.pallas_call(
        _dwconv_ln_kernel,
        grid=(_B, _NWIN),
        in_specs=[
            pl.BlockSpec((1, _SR, _W, _C), _up),
            pl.BlockSpec((1, _SR, _W, _C), _cn),
            pl.BlockSpec((1, _SR, _W, _C), _dn),
            pl.BlockSpec((3, 3, _C), lambda b, j: (0, 0, 0)),
            pl.BlockSpec((1, _C), lambda b, j: (0, 0)),
            pl.BlockSpec((1, _C), lambda b, j: (0, 0)),
            pl.BlockSpec((1, _C), lambda b, j: (0, 0)),
        ],
        out_specs=[_strip(), _strip()],
        out_shape=[
            jax.ShapeDtypeStruct((_B, _H, _W, _C), f32),
            jax.ShapeDtypeStruct((_B, _H, _W, _C), jnp.bfloat16),
        ],
    )(x_bhwc, x_bhwc, x_bhwc, w3, pos_b.reshape(1, _C),
      ln1_g.reshape(1, _C), ln1_b.reshape(1, _C))

    xw = xn.reshape(_B, _NWIN, 8, _NWIN, 8, _C).transpose(0, 1, 3, 2, 4, 5)
    xw = xw.reshape(_B, _P2, _HW, _C)

    wqkv = jnp.concatenate([wq, wkv], axis=1).astype(jnp.bfloat16)
    q, k, v, qm, km = pl.pallas_call(
        _qkv_kernel,
        grid=(_B, _P2 // 7),
        in_specs=[
            pl.BlockSpec((1, 7, _HW, _C), lambda b, j: (b, j, 0, 0)),
            pl.BlockSpec((_C, 3 * _C), lambda b, j: (0, 0)),
        ],
        out_specs=[
            pl.BlockSpec((1, 7, _HW, _C), lambda b, j: (b, j, 0, 0)),
            pl.BlockSpec((1, 7, _HW, _C), lambda b, j: (b, j, 0, 0)),
            pl.BlockSpec((1, 7, _HW, _C), lambda b, j: (b, j, 0, 0)),
            pl.BlockSpec((1, 7, 1, _C), lambda b, j: (b, j, 0, 0)),
            pl.BlockSpec((1, 7, 1, _C), lambda b, j: (b, j, 0, 0)),
        ],
        out_shape=[
            jax.ShapeDtypeStruct((_B, _P2, _HW, _C), jnp.bfloat16),
            jax.ShapeDtypeStruct((_B, _P2, _HW, _C), jnp.bfloat16),
            jax.ShapeDtypeStruct((_B, _P2, _HW, _C), jnp.bfloat16),
            jax.ShapeDtypeStruct((_B, _P2, 1, _C), f32),
            jax.ShapeDtypeStruct((_B, _P2, 1, _C), f32),
        ],
    )(xw, wqkv)

    top_idx = pl.pallas_call(
        _route_kernel,
        grid=(_B,),
        in_specs=[
            pl.BlockSpec((1, _P2, 1, _C), lambda b: (b, 0, 0, 0)),
            pl.BlockSpec((1, _P2, 1, _C), lambda b: (b, 0, 0, 0)),
        ],
        out_specs=pl.BlockSpec((1, _P2, _TOPK), lambda b: (b, 0, 0)),
        out_shape=jax.ShapeDtypeStruct((_B, _P2, _TOPK), jnp.int32),
    )(qm, km)

    attn_out = pl.pallas_call(
        _attn_kernel,
        grid_spec=pltpu.PrefetchScalarGridSpec(
            num_scalar_prefetch=1,
            grid=(_B, _P2),
            in_specs=[
                pl.BlockSpec((1, 1, _HW, _C), lambda b, i, idx: (b, i, 0, 0)),
                pl.BlockSpec((1, _P2, _HW, _C), lambda b, i, idx: (b, 0, 0, 0)),
                pl.BlockSpec((1, _P2, _HW, _C), lambda b, i, idx: (b, 0, 0, 0)),
            ],
            out_specs=pl.BlockSpec((1, 1, _HW, _C), lambda b, i, idx: (b, i, 0, 0)),
            scratch_shapes=[
                pltpu.VMEM((_TOPK * _HW, _C), jnp.bfloat16),
                pltpu.VMEM((_TOPK * _HW, _C), jnp.bfloat16),
                pltpu.VMEM((_NHEADS * _HW, _TOPK * _HW), jnp.float32),
                pltpu.VMEM((_NHEADS * _HW, _TOPK * _HW), jnp.bfloat16),
                pltpu.SemaphoreType.DMA((2 * _TOPK,)),
            ],
        ),
        out_shape=jax.ShapeDtypeStruct((_B, _P2, _HW, _C), jnp.bfloat16),
    )(top_idx, q, k, v)

    def _reverse(t):
        t = t.reshape(_B, _NWIN, _NWIN, 8, 8, _C).transpose(0, 1, 3, 2, 4, 5)
        return t.reshape(_B, _H, _W, _C)

    attn_img = _reverse(attn_out)
    v_img = _reverse(v)

    z = pl.pallas_call(
        _lepe_kernel,
        grid=(_B, _NWIN),
        in_specs=[
            pl.BlockSpec((1, _SR, _W, _C), _up),
            pl.BlockSpec((1, _SR, _W, _C), _cn),
            pl.BlockSpec((1, _SR, _W, _C), _dn),
            pl.BlockSpec((1, _SR, _W, _C), _cn),
            pl.BlockSpec((5, 5, _C), lambda b, j: (0, 0, 0)),
            pl.BlockSpec((1, _C), lambda b, j: (0, 0)),
        ],
        out_specs=pl.BlockSpec((1, _SR, _W, _C), _cn),
        out_shape=jax.ShapeDtypeStruct((_B, _H, _W, _C), jnp.bfloat16),
    )(v_img, v_img, v_img, attn_img, w5, lepe_b.reshape(1, _C))

    z_t = z.reshape(_NTOK, _C)
    y_t = y.reshape(_NTOK, _C)
    x2 = pl.pallas_call(
        _proj_kernel,
        grid=(_NROW,),
        in_specs=[
            pl.BlockSpec((_ROWS, _C), lambda r: (r, 0)),
            pl.BlockSpec((_ROWS, _C), lambda r: (r, 0)),
            pl.BlockSpec((_C, _C), lambda r: (0, 0)),
        ],
        out_specs=pl.BlockSpec((_ROWS, _C), lambda r: (r, 0)),
        out_shape=jax.ShapeDtypeStruct((_NTOK, _C), f32),
    )(z_t, y_t, wo.astype(jnp.bfloat16))

    out_t = pl.pallas_call(
        _mlp_kernel,
        grid=(_NROW,),
        in_specs=[
            pl.BlockSpec((_ROWS, _C), lambda r: (r, 0)),
            pl.BlockSpec((1, _C), lambda r: (0, 0)),
            pl.BlockSpec((1, _C), lambda r: (0, 0)),
            pl.BlockSpec((_C, _C4), lambda r: (0, 0)),
            pl.BlockSpec((1, _C4), lambda r: (0, 0)),
            pl.BlockSpec((_C4, _C), lambda r: (0, 0)),
            pl.BlockSpec((1, _C), lambda r: (0, 0)),
        ],
        out_specs=pl.BlockSpec((_ROWS, _C), lambda r: (r, 0)),
        out_shape=jax.ShapeDtypeStruct((_NTOK, _C), f32),
    )(x2, ln2_g.reshape(1, _C), ln2_b.reshape(1, _C), mlp_w1.astype(jnp.bfloat16),
      mlp_b1.reshape(1, _C4), mlp_w2.astype(jnp.bfloat16), mlp_b2.reshape(1, _C))

    out = out_t.reshape(_B, _H, _W, _C)
    return jnp.transpose(out, (0, 3, 1, 2))


# speedup vs baseline: 2.6017x; 1.0744x over previous
import jax
import jax.numpy as jnp
from jax.experimental import pallas as pl
from jax.experimental.pallas import tpu as pltpu

_B, _C, _H, _W = 2, 768, 56, 56
_NWIN = 7
_TOPK = 8
_NHEADS = 12
_HD = _C // _NHEADS
_P2 = _NWIN * _NWIN
_HW = 64
_C4 = _C * 4
_SR = 8
_SN = _SR * _W


def _strip_conv(up, cur, dn, w_ref, pad, j):
    ksize = 2 * pad + 1
    top = jnp.where(j > 0, up[_SR - pad:], jnp.zeros((pad, _W, _C), up.dtype))
    bot = jnp.where(j < _NWIN - 1, dn[:pad], jnp.zeros((pad, _W, _C), dn.dtype))
    xv = jnp.concatenate([top, cur, bot], axis=0).astype(jnp.float32)
    xp = jnp.pad(xv, ((0, 0), (pad, pad), (0, 0)))
    acc = jnp.zeros((_SR, _W, _C), jnp.float32)
    for dh in range(ksize):
        for dw in range(ksize):
            wv = w_ref[dh, dw, :].reshape(1, 1, _C)
            acc = acc + xp[dh:dh + _SR, dw:dw + _W, :] * wv
    return acc


def _dwconv_ln_kernel(xu_ref, xc_ref, xd_ref, w_ref, pb_ref, g_ref, b_ref,
                      y_ref, xn_ref):
    j = pl.program_id(1)
    acc = _strip_conv(xu_ref[0], xc_ref[0], xd_ref[0], w_ref, 1, j)
    y = xc_ref[0] + acc + pb_ref[0].reshape(1, 1, _C)
    y_ref[0] = y
    mu = jnp.mean(y, axis=-1, keepdims=True)
    var = jnp.mean((y - mu) ** 2, axis=-1, keepdims=True)
    xn = (y - mu) / jnp.sqrt(var + 1e-6)
    xn = xn * g_ref[0].reshape(1, 1, _C) + b_ref[0].reshape(1, 1, _C)
    xn_ref[0] = xn.astype(jnp.bfloat16)


def _win(t):
    return t.reshape(_SR, _NWIN, 8, _C).transpose(1, 0, 2, 3).reshape(_NWIN, _HW, _C)


def _unwin(t):
    return t.reshape(_NWIN, _SR, 8, _C).transpose(1, 0, 2, 3).reshape(_SR, _W, _C)


def _qkv_kernel(xn_ref, w_ref, q_ref, k_ref, v_ref, vimg_ref, qm_ref, km_ref):
    xb = _win(xn_ref[0]).reshape(_SN, _C)
    qkv = jnp.dot(xb, w_ref[...], preferred_element_type=jnp.float32)
    q = qkv[:, :_C]
    k = qkv[:, _C:2 * _C]
    v = qkv[:, 2 * _C:]
    q_ref[0] = q.reshape(_NWIN, _HW, _C).astype(jnp.bfloat16)
    k_ref[0] = k.reshape(_NWIN, _HW, _C).astype(jnp.bfloat16)
    vw = v.reshape(_NWIN, _HW, _C).astype(jnp.bfloat16)
    v_ref[0] = vw
    vimg_ref[0] = _unwin(vw)
    qm_ref[0] = q.reshape(_NWIN, _HW, _C).mean(axis=1, keepdims=True)
    km_ref[0] = k.reshape(_NWIN, _HW, _C).mean(axis=1, keepdims=True)


def _route_kernel(qm_ref, km_ref, idx_ref):
    qw = qm_ref[0, :, 0, :]
    kw = km_ref[0, :, 0, :]
    adj = jax.lax.dot_general(qw, kw, (((1,), (1,)), ((), ())),
                              preferred_element_type=jnp.float32)
    col = jax.lax.broadcasted_iota(jnp.int32, (_P2, _P2), 1)
    idxs = []
    a = adj
    for _ in range(_TOPK):
        m = jnp.max(a, axis=1, keepdims=True)
        idx = jnp.min(jnp.where(a >= m, col, _P2 * 2), axis=1)
        idxs.append(idx)
        a = jnp.where(col == idx[:, None], -jnp.inf, a)
    idx_ref[0] = jnp.stack(idxs, axis=1).astype(jnp.int32)


def _attn_kernel(idx_ref, q_ref, k_ref, v_ref, o_ref, kall_ref, vall_ref,
                 s_ref, p_ref, sems):
    b = pl.program_id(0)
    i = pl.program_id(1)
    copies = []
    for t in range(_TOPK):
        w = idx_ref[b, i, t]
        for src, dst, s in ((k_ref, kall_ref, 0), (v_ref, vall_ref, 1)):
            c = pltpu.make_async_copy(
                src.at[0, w], dst.at[pl.ds(t * _HW, _HW), :], sems.at[2 * t + s])
            c.start()
            copies.append(c)
    for c in copies:
        c.wait()
    q = q_ref[0, 0]
    scale = _HD ** -0.5
    for h in range(_NHEADS):
        sl = slice(h * _HD, (h + 1) * _HD)
        s_ref[pl.ds(h * _HW, _HW), :] = jax.lax.dot_general(
            q[:, sl], kall_ref[:, sl], (((1,), (1,)), ((), ())),
            preferred_element_type=jnp.float32)
    s = s_ref[...] * scale
    s = s - jnp.max(s, axis=1, keepdims=True)
    e = jnp.exp(s)
    p_ref[...] = (e / jnp.sum(e, axis=1, keepdims=True)).astype(jnp.bfloat16)
    outs = []
    for h in range(_NHEADS):
        sl = slice(h * _HD, (h + 1) * _HD)
        outs.append(jnp.dot(p_ref[pl.ds(h * _HW, _HW), :], vall_ref[:, sl],
                            preferred_element_type=jnp.float32))
    o_ref[0] = jnp.concatenate(outs, axis=1).astype(jnp.bfloat16).reshape(8, 8, _C)


def _tail_kernel(vu_ref, vc_ref, vd_ref, a_ref, y_ref, w5_ref, lb_ref,
                 wo_ref, g_ref, b_ref, w1_ref, b1_ref, w2_ref, b2_ref, o_ref):
    j = pl.program_id(1)
    acc = _strip_conv(vu_ref[0], vc_ref[0], vd_ref[0], w5_ref, 2, j)
    z = a_ref[0].astype(jnp.float32) + acc + lb_ref[0].reshape(1, 1, _C)
    z = z.astype(jnp.bfloat16).reshape(_SN, _C)
    x2 = y_ref[0].reshape(_SN, _C) + jnp.dot(z, wo_ref[...],
                                             preferred_element_type=jnp.float32)
    mu = jnp.mean(x2, axis=-1, keepdims=True)
    var = jnp.mean((x2 - mu) ** 2, axis=-1, keepdims=True)
    xn = (x2 - mu) / jnp.sqrt(var + 1e-6) * g_ref[0].reshape(1, _C) + b_ref[0].reshape(1, _C)
    h1 = jnp.dot(xn.astype(jnp.bfloat16), w1_ref[...],
                 preferred_element_type=jnp.float32) + b1_ref[0].reshape(1, _C4)
    h1 = 0.5 * h1 * (1.0 + jax.lax.erf(h1 * (2.0 ** -0.5)))
    out = x2 + jnp.dot(h1.astype(jnp.bfloat16), w2_ref[...],
                       preferred_element_type=jnp.float32) + b2_ref[0].reshape(1, _C)
    o_ref[0] = out.reshape(_SR, _W, _C)


def kernel(x, pos_w, pos_b, ln1_g, ln1_b, wq, wkv, wo, lepe_w, lepe_b,
           ln2_g, ln2_b, mlp_w1, mlp_b1, mlp_w2, mlp_b2):
    f32 = jnp.float32
    bf16 = jnp.bfloat16
    x_bhwc = jnp.transpose(x, (0, 2, 3, 1))
    w3 = jnp.transpose(pos_w[:, 0], (1, 2, 0))
    w5 = jnp.transpose(lepe_w[:, 0], (1, 2, 0))

    _up = lambda b, j: (b, jnp.maximum(j - 1, 0), 0, 0)
    _cn = lambda b, j: (b, j, 0, 0)
    _dn = lambda b, j: (b, jnp.minimum(j + 1, _NWIN - 1), 0, 0)
    _strip = lambda: pl.BlockSpec((1, _SR, _W, _C), _cn)
    _vec = lambda n=_C: pl.BlockSpec((1, n), lambda b, j: (0, 0))

    y, xn = pl.pallas_call(
        _dwconv_ln_kernel,
        grid=(_B, _NWIN),
        in_specs=[
            pl.BlockSpec((1, _SR, _W, _C), _up),
            pl.BlockSpec((1, _SR, _W, _C), _cn),
            pl.BlockSpec((1, _SR, _W, _C), _dn),
            pl.BlockSpec((3, 3, _C), lambda b, j: (0, 0, 0)),
            _vec(), _vec(), _vec(),
        ],
        out_specs=[_strip(), _strip()],
        out_shape=[
            jax.ShapeDtypeStruct((_B, _H, _W, _C), f32),
            jax.ShapeDtypeStruct((_B, _H, _W, _C), bf16),
        ],
    )(x_bhwc, x_bhwc, x_bhwc, w3, pos_b.reshape(1, _C),
      ln1_g.reshape(1, _C), ln1_b.reshape(1, _C))

    wqkv = jnp.concatenate([wq, wkv], axis=1).astype(bf16)
    _winblk = pl.BlockSpec((1, _NWIN, _HW, _C), _cn)
    q, k, v, v_img, qm, km = pl.pallas_call(
        _qkv_kernel,
        grid=(_B, _NWIN),
        in_specs=[
            _strip(),
            pl.BlockSpec((_C, 3 * _C), lambda b, j: (0, 0)),
        ],
        out_specs=[
            _winblk, _winblk, _winblk,
            _strip(),
            pl.BlockSpec((1, _NWIN, 1, _C), _cn),
            pl.BlockSpec((1, _NWIN, 1, _C), _cn),
        ],
        out_shape=[
            jax.ShapeDtypeStruct((_B, _P2, _HW, _C), bf16),
            jax.ShapeDtypeStruct((_B, _P2, _HW, _C), bf16),
            jax.ShapeDtypeStruct((_B, _P2, _HW, _C), bf16),
            jax.ShapeDtypeStruct((_B, _H, _W, _C), bf16),
            jax.ShapeDtypeStruct((_B, _P2, 1, _C), f32),
            jax.ShapeDtypeStruct((_B, _P2, 1, _C), f32),
        ],
    )(xn, wqkv)

    top_idx = pl.pallas_call(
        _route_kernel,
        grid=(_B,),
        in_specs=[
            pl.BlockSpec((1, _P2, 1, _C), lambda b: (b, 0, 0, 0)),
            pl.BlockSpec((1, _P2, 1, _C), lambda b: (b, 0, 0, 0)),
        ],
        out_specs=pl.BlockSpec((1, _P2, _TOPK), lambda b: (b, 0, 0)),
        out_shape=jax.ShapeDtypeStruct((_B, _P2, _TOPK), jnp.int32),
    )(qm, km)

    attn_img = pl.pallas_call(
        _attn_kernel,
        grid_spec=pltpu.PrefetchScalarGridSpec(
            num_scalar_prefetch=1,
            grid=(_B, _P2),
            in_specs=[
                pl.BlockSpec((1, 1, _HW, _C), lambda b, i, idx: (b, i, 0, 0)),
                pl.BlockSpec((1, _P2, _HW, _C), lambda b, i, idx: (b, 0, 0, 0)),
                pl.BlockSpec((1, _P2, _HW, _C), lambda b, i, idx: (b, 0, 0, 0)),
            ],
            out_specs=pl.BlockSpec((1, 8, 8, _C),
                                   lambda b, i, idx: (b, i // _NWIN, i % _NWIN, 0)),
            scratch_shapes=[
                pltpu.VMEM((_TOPK * _HW, _C), bf16),
                pltpu.VMEM((_TOPK * _HW, _C), bf16),
                pltpu.VMEM((_NHEADS * _HW, _TOPK * _HW), f32),
                pltpu.VMEM((_NHEADS * _HW, _TOPK * _HW), bf16),
                pltpu.SemaphoreType.DMA((2 * _TOPK,)),
            ],
        ),
        out_shape=jax.ShapeDtypeStruct((_B, _H, _W, _C), bf16),
    )(top_idx, q, k, v)

    out_img = pl.pallas_call(
        _tail_kernel,
        grid=(_B, _NWIN),
        in_specs=[
            pl.BlockSpec((1, _SR, _W, _C), _up),
            pl.BlockSpec((1, _SR, _W, _C), _cn),
            pl.BlockSpec((1, _SR, _W, _C), _dn),
            _strip(),
            _strip(),
            pl.BlockSpec((5, 5, _C), lambda b, j: (0, 0, 0)),
            _vec(),
            pl.BlockSpec((_C, _C), lambda b, j: (0, 0)),
            _vec(), _vec(),
            pl.BlockSpec((_C, _C4), lambda b, j: (0, 0)),
            _vec(_C4),
            pl.BlockSpec((_C4, _C), lambda b, j: (0, 0)),
            _vec(),
        ],
        out_specs=_strip(),
        out_shape=jax.ShapeDtypeStruct((_B, _H, _W, _C), f32),
    )(v_img, v_img, v_img, attn_img, y, w5, lepe_b.reshape(1, _C),
      wo.astype(bf16), ln2_g.reshape(1, _C), ln2_b.reshape(1, _C),
      mlp_w1.astype(bf16), mlp_b1.reshape(1, _C4), mlp_w2.astype(bf16),
      mlp_b2.reshape(1, _C))

    return jnp.transpose(out_img, (0, 3, 1, 2))
